# baseline (speedup 1.0000x reference)
import functools

import jax
import jax.numpy as jnp
from jax import lax
from jax.experimental import pallas as pl
from jax.experimental.pallas import tpu as pltpu
from jax.experimental.pallas import tpu_sc as plsc

C = 64
NX = 512
NY = 512
NCELLS = NX * NY
N = 30000
NPAD = 30720
NSC = 2
NTILES = 16
NP_T = NPAD // NTILES
CHUNK = 128
NCHUNK = NP_T // CHUNK
WIN = 384
NWIN = NP_T // WIN
SEG = 16384
NSEG = NCELLS // (NSC * SEG)
TRASH = 128
ZROWS = SEG // NTILES
ZBUF = 128


NPART = 4
SEG_P = NSEG // NPART
PCELLS = NSC * SEG_P * SEG


def _sc_scatter(pillar_features, lin_pad, zeros_blk, part):
    nseg_h = SEG_P
    hcells = PCELLS
    hbase = part * PCELLS
    mesh = plsc.VectorSubcoreMesh(core_axis_name="c", subcore_axis_name="s")

    @functools.partial(
        pl.kernel,
        mesh=mesh,
        out_type=jax.ShapeDtypeStruct((hcells, 2 * C), jnp.float32),
        compiler_params=pltpu.CompilerParams(use_tc_tiling_on_sc=False),
        scratch_types=[
            pltpu.VMEM((WIN, C), jnp.float32),
            pltpu.VMEM((WIN, C), jnp.float32),
            pltpu.VMEM((ZBUF, C), jnp.float32),
            pltpu.VMEM((NP_T,), jnp.int32),
            pltpu.VMEM((NCHUNK, CHUNK), jnp.int32),
            pltpu.VMEM_SHARED((SEG + TRASH, C), jnp.float32),
            pltpu.SemaphoreType.DMA,
            pltpu.SemaphoreType.DMA,
            pltpu.SemaphoreType.DMA,
            pltpu.SemaphoreType.DMA,
            pltpu.SemaphoreType.DMA,
        ],
    )
    def body(feat_hbm, lin_hbm, zeros_hbm, out_hbm, fbuf0, fbuf1, zbuf,
             lin_v, offs_v, acc_sh, sem_z, sem_s0, sem_s1, sem_c0, sem_c1):
        cid = lax.axis_index("c")
        sid = lax.axis_index("s")
        tbase = sid * NP_T
        fstart = pl.multiple_of(
            jnp.where(sid == NTILES - 1, N - NP_T, tbase), 16)
        fbuf = (fbuf0, fbuf1)
        sem_s = (sem_s0, sem_s1)
        sem_c = (sem_c0, sem_c1)

        pltpu.sync_copy(lin_hbm.at[pl.ds(tbase, NP_T)], lin_v)
        pltpu.sync_copy(zeros_hbm, zbuf)

        def seg_body(p, _):
            lbase = cid * (nseg_h * SEG) + p * SEG
            base = hbase + lbase

            zh = [pltpu.async_copy(
                zbuf, acc_sh.at[pl.ds(sid * ZROWS + q * ZBUF, ZBUF)], sem_z)
                for q in range(ZROWS // ZBUF)]

            def off_body(jr, _):
                for k in range(CHUNK // 16):
                    lin = lin_v[pl.ds(jr * CHUNK + k * 16, 16)]
                    off = lin - base
                    inr = (off >= 0) & (off < SEG)
                    off = jnp.where(inr, off, SEG + (lin & (TRASH - 1)))
                    offs_v[jr, pl.ds(k * 16, 16)] = off
                return 0

            lax.fori_loop(0, NCHUNK, off_body, 0)
            for h in zh:
                h.wait()
            plsc.subcore_barrier()

            sh = [None] * NWIN
            ch = [None] * NWIN
            sh[0] = pltpu.async_copy(
                feat_hbm.at[pl.ds(fstart, WIN)], fbuf[0], sem_s[0])
            for w in range(NWIN):
                b = w & 1
                sh[w].wait()
                if w + 1 < NWIN:
                    if w >= 1:
                        for h in ch[w - 1]:
                            h.wait()
                    sh[w + 1] = pltpu.async_copy(
                        feat_hbm.at[pl.ds(fstart + (w + 1) * WIN, WIN)],
                        fbuf[1 - b], sem_s[1 - b])
                ch[w] = [pltpu.async_copy(
                    fbuf[b].at[pl.ds(h * CHUNK, CHUNK)],
                    acc_sh.at[offs_v.at[w * (WIN // CHUNK) + h]],
                    sem_c[b], add=True)
                    for h in range(WIN // CHUNK)]
            for h in ch[NWIN - 2] + ch[NWIN - 1]:
                h.wait()
            plsc.subcore_barrier()

            pltpu.sync_copy(acc_sh.at[pl.ds(sid * ZROWS, ZROWS)],
                            out_hbm.at[pl.ds(lbase + sid * ZROWS, ZROWS),
                                       pl.ds(0, C)])
            return 0

        lax.fori_loop(0, nseg_h, seg_body, 0)

    return body(pillar_features, lin_pad, zeros_blk)


TROWS = 8


def _tc_transpose_half(flat_half, eye, part, prev):
    def tkernel(x_ref, e_ref, *rest):
        o_ref = rest[-1]
        for r in range(TROWS):
            o_ref[:, r, :] = lax.dot_general(
                x_ref[pl.ds(r * NX, NX), :C], e_ref[...],
                (((0,), (0,)), ((), ())),
                preferred_element_type=jnp.float32)

    ny_h = flat_half.shape[0] // NX
    hb = part * ((PCELLS // NX) // TROWS)
    in_specs = [pl.BlockSpec((TROWS * NX, 2 * C), lambda i: (i, 0)),
                pl.BlockSpec((NX, NX), lambda i: (0, 0))]
    args = [flat_half, eye]
    aliases = {}
    if prev is not None:
        in_specs.append(pl.BlockSpec(memory_space=pl.ANY))
        args.append(prev)
        aliases = {2: 0}
    return pl.pallas_call(
        tkernel,
        grid=(ny_h // TROWS,),
        in_specs=in_specs,
        out_specs=pl.BlockSpec((C, TROWS, NX), lambda i: (0, hb + i, 0)),
        out_shape=jax.ShapeDtypeStruct((C, NY, NX), jnp.float32),
        input_output_aliases=aliases,
    )(*args)


def kernel(pillar_features, pillar_coords):
    coords = pillar_coords.astype(jnp.int32)
    lin = (coords * jnp.array([1, NX], jnp.int32)).sum(axis=1)
    t15 = jnp.where(jnp.arange(NP_T) < NPAD - N, -1, lin[N - NP_T:])
    lin_all = jnp.concatenate([lin[:NP_T * (NTILES - 1)], t15])
    zeros_blk = jnp.zeros((ZBUF, C), jnp.float32)
    eye = jnp.eye(NX, dtype=jnp.float32)
    flats = [_sc_scatter(pillar_features, lin_all, zeros_blk, k)
             for k in range(NPART)]
    bev = None
    for k in range(NPART):
        bev = _tc_transpose_half(flats[k], eye, k, bev)
    return bev.reshape(1, C, NY, NX)


# speedup vs baseline: 1.0140x; 1.0140x over previous
import functools

import jax
import jax.numpy as jnp
from jax import lax
from jax.experimental import pallas as pl
from jax.experimental.pallas import tpu as pltpu
from jax.experimental.pallas import tpu_sc as plsc

C = 64
NX = 512
NY = 512
NCELLS = NX * NY
N = 30000
NPAD = 30720
NSC = 2
NTILES = 16
NP_T = NPAD // NTILES
CHUNK = 128
NCHUNK = NP_T // CHUNK
WIN = 384
NWIN = NP_T // WIN
SEG = 16384
NSEG = NCELLS // (NSC * SEG)
TRASH = 128
ZROWS = SEG // NTILES
ZBUF = 128


NPART = 4
SEG_P = NSEG // NPART
PCELLS = NSC * SEG_P * SEG


def _sc_scatter(pillar_features, lin_pad, zeros_blk, part):
    nseg_h = SEG_P
    hcells = PCELLS
    hbase = part * PCELLS
    mesh = plsc.VectorSubcoreMesh(core_axis_name="c", subcore_axis_name="s")

    @functools.partial(
        pl.kernel,
        mesh=mesh,
        out_type=jax.ShapeDtypeStruct((hcells, 2 * C), jnp.float32),
        compiler_params=pltpu.CompilerParams(use_tc_tiling_on_sc=False),
        scratch_types=[
            pltpu.VMEM((WIN, C), jnp.float32),
            pltpu.VMEM((WIN, C), jnp.float32),
            pltpu.VMEM((ZBUF, C), jnp.float32),
            pltpu.VMEM((NP_T,), jnp.int32),
            pltpu.VMEM((NCHUNK, CHUNK), jnp.int32),
            pltpu.VMEM_SHARED((SEG + TRASH, C), jnp.float32),
            pltpu.SemaphoreType.DMA,
            pltpu.SemaphoreType.DMA,
            pltpu.SemaphoreType.DMA,
            pltpu.SemaphoreType.DMA,
            pltpu.SemaphoreType.DMA,
        ],
    )
    def body(feat_hbm, lin_hbm, zeros_hbm, out_hbm, fbuf0, fbuf1, zbuf,
             lin_v, offs_v, acc_sh, sem_z, sem_s0, sem_s1, sem_c0, sem_c1):
        cid = lax.axis_index("c")
        sid = lax.axis_index("s")
        tbase = sid * NP_T
        fstart = pl.multiple_of(
            jnp.where(sid == NTILES - 1, N - NP_T, tbase), 16)
        fbuf = (fbuf0, fbuf1)
        sem_s = (sem_s0, sem_s1)
        sem_c = (sem_c0, sem_c1)

        pltpu.sync_copy(lin_hbm.at[pl.ds(tbase, NP_T)], lin_v)
        pltpu.sync_copy(zeros_hbm, zbuf)

        def seg_body(p, _):
            lbase = cid * (nseg_h * SEG) + p * SEG
            base = hbase + lbase

            zh = [pltpu.async_copy(
                zbuf, acc_sh.at[pl.ds(sid * ZROWS + q * ZBUF, ZBUF)], sem_z)
                for q in range(ZROWS // ZBUF)]

            def off_body(jr, _):
                for k in range(CHUNK // 16):
                    lin = lin_v[pl.ds(jr * CHUNK + k * 16, 16)]
                    off = lin - base
                    inr = (off >= 0) & (off < SEG)
                    off = jnp.where(inr, off, SEG + (lin & (TRASH - 1)))
                    offs_v[jr, pl.ds(k * 16, 16)] = off
                return 0

            lax.fori_loop(0, NCHUNK, off_body, 0)
            for h in zh:
                h.wait()
            plsc.subcore_barrier()

            sh = [None] * NWIN
            ch = [None] * NWIN
            sh[0] = pltpu.async_copy(
                feat_hbm.at[pl.ds(fstart, WIN)], fbuf[0], sem_s[0])
            for w in range(NWIN):
                b = w & 1
                sh[w].wait()
                if w + 1 < NWIN:
                    if w >= 1:
                        for h in ch[w - 1]:
                            h.wait()
                    sh[w + 1] = pltpu.async_copy(
                        feat_hbm.at[pl.ds(fstart + (w + 1) * WIN, WIN)],
                        fbuf[1 - b], sem_s[1 - b])
                ch[w] = [pltpu.async_copy(
                    fbuf[b].at[pl.ds(h * CHUNK, CHUNK)],
                    acc_sh.at[offs_v.at[w * (WIN // CHUNK) + h]],
                    sem_c[b], add=True)
                    for h in range(WIN // CHUNK)]
            for h in ch[NWIN - 2] + ch[NWIN - 1]:
                h.wait()
            plsc.subcore_barrier()

            pltpu.sync_copy(acc_sh.at[pl.ds(sid * ZROWS, ZROWS)],
                            out_hbm.at[pl.ds(lbase + sid * ZROWS, ZROWS),
                                       pl.ds(0, C)])
            return 0

        lax.fori_loop(0, nseg_h, seg_body, 0)

    return body(pillar_features, lin_pad, zeros_blk)


TROWS = 8


def _tc_transpose_half(flat_half, eye, part, prev):
    def tkernel(x_ref, e_ref, *rest):
        o_ref = rest[-1]
        for r in range(TROWS):
            o_ref[:, r, :] = lax.dot_general(
                x_ref[pl.ds(r * NX, NX), :C], e_ref[...],
                (((0,), (0,)), ((), ())),
                preferred_element_type=jnp.float32)

    ny_h = flat_half.shape[0] // NX
    hb = part * ((PCELLS // NX) // TROWS)
    in_specs = [pl.BlockSpec((TROWS * NX, 2 * C), lambda i: (i, 0)),
                pl.BlockSpec((NX, NX), lambda i: (0, 0))]
    args = [flat_half, eye]
    aliases = {}
    if prev is not None:
        in_specs.append(pl.BlockSpec(memory_space=pl.ANY))
        args.append(prev)
        aliases = {2: 0}
    return pl.pallas_call(
        tkernel,
        grid=(ny_h // TROWS,),
        in_specs=in_specs,
        out_specs=pl.BlockSpec((C, TROWS, NX), lambda i: (0, hb + i, 0)),
        out_shape=jax.ShapeDtypeStruct((C, NY, NX), jnp.float32),
        input_output_aliases=aliases,
    )(*args)


def kernel(pillar_features, pillar_coords):
    coords = pillar_coords.astype(jnp.int32)
    lin = coords[:, 1] * NX + coords[:, 0]
    t15 = jnp.where(jnp.arange(NP_T) < NPAD - N, -1, lin[N - NP_T:])
    lin_all = jnp.concatenate([lin[:NP_T * (NTILES - 1)], t15])
    zeros_blk = jnp.zeros((ZBUF, C), jnp.float32)
    eye = jnp.eye(NX, dtype=jnp.float32)
    flats = [_sc_scatter(pillar_features, lin_all, zeros_blk, k)
             for k in range(NPART)]
    bev = None
    for k in range(NPART):
        bev = _tc_transpose_half(flats[k], eye, k, bev)
    return bev.reshape(1, C, NY, NX)
